# Initial kernel scaffold; baseline (speedup 1.0000x reference)
import functools

import jax
import jax.numpy as jnp
from jax import lax
from jax.experimental import pallas as pl
from jax.experimental.pallas import tpu as pltpu
from jax.experimental.pallas import tpu_sc as plsc

N = 10000
E = 320000
D = 128
H = 128
HH = H // 2

NP = 10240
EC = 80
ROWS = E // EC
RB = 25
NT = 16
RPT = ROWS // NT
NBLK = RPT // RB
NPT = NP // NT
NRT = N // NT

_MAGIC = jnp.int32(0x5F3759DF)


def _rsqrt_newton(x):
  i = plsc.bitcast(x, jnp.int32)
  y = plsc.bitcast(_MAGIC - lax.shift_right_logical(i, 1), jnp.float32)
  for _ in range(3):
    y = y * (1.5 - 0.5 * x * y * y)
  return y


def _prep_body(src2d, dst2d, ew2d, d2_out, norm_out,
               srcbuf, dstbuf, ewbuf, padbuf, degbuf, dinvbuf, d2buf,
               dinvv, normbuf, shared_deg, shared_dinv):
  sid = lax.axis_index("s")
  iota16 = lax.iota(jnp.int32, 16)
  zeros16i = jnp.zeros((16,), jnp.int32)
  zeros16f = jnp.zeros((16,), jnp.float32)

  def z_deg(i, _):
    degbuf[i, :] = zeros16f
    return _
  lax.fori_loop(0, NPT, z_deg, None)
  pltpu.sync_copy(degbuf, shared_deg.at[pl.ds(sid * NPT, NPT)])

  def z_pad(i, _):
    padbuf[i, :] = zeros16f
    return _
  lax.fori_loop(0, EC, z_pad, None)
  plsc.subcore_barrier()

  def deg_block(b, _):
    row0 = sid * RPT + b * RB
    pltpu.sync_copy(dst2d.at[pl.ds(row0, RB)], dstbuf)
    pltpu.sync_copy(ew2d.at[pl.ds(row0, RB)], ewbuf)

    def deg_chunk(j, _):
      for g in range(EC // 16):
        ewv = ewbuf[j, pl.ds(g * 16, 16)]
        plsc.store_scatter(padbuf, [iota16 + g * 16, zeros16i], ewv)
      pltpu.sync_copy(padbuf, shared_deg.at[dstbuf.at[j]], add=True)
      return _
    lax.fori_loop(0, RB, deg_chunk, None)
    return _
  lax.fori_loop(0, NBLK, deg_block, None)
  plsc.subcore_barrier()

  pltpu.sync_copy(shared_deg.at[pl.ds(sid * NPT, NPT)], degbuf)

  def dinv_group(g, _):
    deg = plsc.load_gather(degbuf, [iota16 + g * 16, zeros16i]) + 1.0
    y = _rsqrt_newton(deg)
    dinvbuf[pl.ds(g * 16, 16)] = y
    d2buf[pl.ds(g * 16, 16)] = y * y
    return _
  lax.fori_loop(0, NPT // 16, dinv_group, None)
  pltpu.sync_copy(dinvbuf, shared_dinv.at[pl.ds(sid * NPT, NPT)])
  pltpu.sync_copy(d2buf, d2_out.at[pl.ds(sid * NPT, NPT)])
  plsc.subcore_barrier()

  pltpu.sync_copy(shared_dinv, dinvv)

  def norm_block(b, _):
    row0 = sid * RPT + b * RB
    pltpu.sync_copy(src2d.at[pl.ds(row0, RB)], srcbuf)
    pltpu.sync_copy(dst2d.at[pl.ds(row0, RB)], dstbuf)
    pltpu.sync_copy(ew2d.at[pl.ds(row0, RB)], ewbuf)

    def norm_chunk(j, _):
      for g in range(EC // 16):
        srcv = srcbuf[j, pl.ds(g * 16, 16)]
        dstv = dstbuf[j, pl.ds(g * 16, 16)]
        ewv = ewbuf[j, pl.ds(g * 16, 16)]
        dsv = plsc.load_gather(dinvv, [srcv])
        ddv = plsc.load_gather(dinvv, [dstv])
        normbuf[j, pl.ds(g * 16, 16)] = dsv * ewv * ddv
      return _
    lax.fori_loop(0, RB, norm_chunk, None)
    pltpu.sync_copy(normbuf, norm_out.at[pl.ds(row0, RB)])
    return _
  lax.fori_loop(0, NBLK, norm_block, None)


def _prep(src2d, dst2d, ew2d):
  mesh = plsc.VectorSubcoreMesh(
      core_axis_name="c", subcore_axis_name="s", num_cores=1, num_subcores=NT)
  f = pl.kernel(
      _prep_body,
      out_type=(jax.ShapeDtypeStruct((NP,), jnp.float32),
                jax.ShapeDtypeStruct((ROWS, EC), jnp.float32)),
      mesh=mesh,
      scratch_types=[
          pltpu.VMEM((RB, EC), jnp.int32),
          pltpu.VMEM((RB, EC), jnp.int32),
          pltpu.VMEM((RB, EC), jnp.float32),
          pltpu.VMEM((EC, 16), jnp.float32),
          pltpu.VMEM((NPT, 16), jnp.float32),
          pltpu.VMEM((NPT,), jnp.float32),
          pltpu.VMEM((NPT,), jnp.float32),
          pltpu.VMEM((NP,), jnp.float32),
          pltpu.VMEM((RB, EC), jnp.float32),
          pltpu.VMEM_SHARED((NP, 16), jnp.float32),
          pltpu.VMEM_SHARED((NP,), jnp.float32),
      ])
  return f(src2d, dst2d, ew2d)


def _msg_body(h_lo, h_hi, src2d, dst2d, norm2d, acc_lo, acc_hi,
              srcbuf, dstbuf, normbuf, rows_v, zbuf, shared_acc):
  cid = lax.axis_index("c")
  sid = lax.axis_index("s")
  zeros16f = jnp.zeros((16,), jnp.float32)

  def z_row(i, _):
    for q in range(HH // 16):
      zbuf[i, pl.ds(q * 16, 16)] = zeros16f
    return _
  lax.fori_loop(0, 125, z_row, None)
  for k in range(NRT // 125):
    pltpu.sync_copy(zbuf, shared_acc.at[pl.ds(sid * NRT + k * 125, 125)])
  plsc.subcore_barrier()

  def msg_block(b, _):
    row0 = sid * RPT + b * RB
    pltpu.sync_copy(src2d.at[pl.ds(row0, RB)], srcbuf)
    pltpu.sync_copy(dst2d.at[pl.ds(row0, RB)], dstbuf)
    pltpu.sync_copy(norm2d.at[pl.ds(row0, RB)], normbuf)

    def msg_chunk(j, _):
      @pl.when(cid == 0)
      def _g0():
        pltpu.sync_copy(h_lo.at[srcbuf.at[j]], rows_v)

      @pl.when(cid == 1)
      def _g1():
        pltpu.sync_copy(h_hi.at[srcbuf.at[j]], rows_v)

      def scale_edge(e, _):
        w = normbuf[j, e]
        for q in range(HH // 16):
          rows_v[e, pl.ds(q * 16, 16)] = rows_v[e, pl.ds(q * 16, 16)] * w
        return _
      lax.fori_loop(0, EC, scale_edge, None)

      pltpu.sync_copy(rows_v, shared_acc.at[dstbuf.at[j]], add=True)
      return _
    lax.fori_loop(0, RB, msg_chunk, None)
    return _
  lax.fori_loop(0, NBLK, msg_block, None)
  plsc.subcore_barrier()

  @pl.when(cid == 0)
  def _w0():
    pltpu.sync_copy(shared_acc.at[pl.ds(sid * NRT, NRT)],
                    acc_lo.at[pl.ds(sid * NRT, NRT)])

  @pl.when(cid == 1)
  def _w1():
    pltpu.sync_copy(shared_acc.at[pl.ds(sid * NRT, NRT)],
                    acc_hi.at[pl.ds(sid * NRT, NRT)])


def _msg(h_lo, h_hi, src2d, dst2d, norm2d):
  mesh = plsc.VectorSubcoreMesh(
      core_axis_name="c", subcore_axis_name="s", num_cores=2, num_subcores=NT)
  f = pl.kernel(
      _msg_body,
      out_type=(jax.ShapeDtypeStruct((N, HH), jnp.float32),
                jax.ShapeDtypeStruct((N, HH), jnp.float32)),
      mesh=mesh,
      scratch_types=[
          pltpu.VMEM((RB, EC), jnp.int32),
          pltpu.VMEM((RB, EC), jnp.int32),
          pltpu.VMEM((RB, EC), jnp.float32),
          pltpu.VMEM((EC, HH), jnp.float32),
          pltpu.VMEM((125, HH), jnp.float32),
          pltpu.VMEM_SHARED((N, HH), jnp.float32),
      ])
  return f(h_lo, h_hi, src2d, dst2d, norm2d)


TBLK = 1000
TNB = N // TBLK


def _mm1_body(x_ref, w_ref, lo_ref, hi_ref):
  h = jnp.dot(x_ref[...], w_ref[...], preferred_element_type=jnp.float32)
  lo_ref[...] = h[:, :HH]
  hi_ref[...] = h[:, HH:]


def _mm1(x, W):
  return pl.pallas_call(
      _mm1_body,
      grid=(TNB,),
      in_specs=[
          pl.BlockSpec((TBLK, D), lambda i: (i, 0)),
          pl.BlockSpec((D, H), lambda i: (0, 0)),
      ],
      out_specs=[
          pl.BlockSpec((TBLK, HH), lambda i: (i, 0)),
          pl.BlockSpec((TBLK, HH), lambda i: (i, 0)),
      ],
      out_shape=[jax.ShapeDtypeStruct((N, HH), jnp.float32),
                 jax.ShapeDtypeStruct((N, HH), jnp.float32)],
  )(x, W)


def _mid_body(al_ref, ah_ref, hl_ref, hh_ref, d2_ref, b_ref, w_ref,
              lo_ref, hi_ref):
  d2 = d2_ref[...]
  z = jnp.concatenate(
      [al_ref[...] + d2 * hl_ref[...], ah_ref[...] + d2 * hh_ref[...]],
      axis=1) + b_ref[...]
  h2 = jnp.dot(z, w_ref[...], preferred_element_type=jnp.float32)
  lo_ref[...] = h2[:, :HH]
  hi_ref[...] = h2[:, HH:]


def _mid(acc_lo, acc_hi, h_lo, h_hi, d2, b, W):
  return pl.pallas_call(
      _mid_body,
      grid=(TNB,),
      in_specs=[
          pl.BlockSpec((TBLK, HH), lambda i: (i, 0)),
          pl.BlockSpec((TBLK, HH), lambda i: (i, 0)),
          pl.BlockSpec((TBLK, HH), lambda i: (i, 0)),
          pl.BlockSpec((TBLK, HH), lambda i: (i, 0)),
          pl.BlockSpec((TBLK, 1), lambda i: (i, 0)),
          pl.BlockSpec((1, H), lambda i: (0, 0)),
          pl.BlockSpec((H, H), lambda i: (0, 0)),
      ],
      out_specs=[
          pl.BlockSpec((TBLK, HH), lambda i: (i, 0)),
          pl.BlockSpec((TBLK, HH), lambda i: (i, 0)),
      ],
      out_shape=[jax.ShapeDtypeStruct((N, HH), jnp.float32),
                 jax.ShapeDtypeStruct((N, HH), jnp.float32)],
  )(acc_lo, acc_hi, h_lo, h_hi, d2, b, W)


def _final_body(al_ref, ah_ref, hl_ref, hh_ref, d2_ref, b_ref, wp_ref,
                bp_ref, z_ref, p_ref):
  d2 = d2_ref[...]
  z = jnp.concatenate(
      [al_ref[...] + d2 * hl_ref[...], ah_ref[...] + d2 * hh_ref[...]],
      axis=1) + b_ref[...]
  z_ref[...] = z
  p_ref[...] = jnp.dot(z, wp_ref[...],
                       preferred_element_type=jnp.float32) + bp_ref[...]


def _final(acc_lo, acc_hi, h_lo, h_hi, d2, b, Wp, bp):
  return pl.pallas_call(
      _final_body,
      grid=(TNB,),
      in_specs=[
          pl.BlockSpec((TBLK, HH), lambda i: (i, 0)),
          pl.BlockSpec((TBLK, HH), lambda i: (i, 0)),
          pl.BlockSpec((TBLK, HH), lambda i: (i, 0)),
          pl.BlockSpec((TBLK, HH), lambda i: (i, 0)),
          pl.BlockSpec((TBLK, 1), lambda i: (i, 0)),
          pl.BlockSpec((1, H), lambda i: (0, 0)),
          pl.BlockSpec((H, H), lambda i: (0, 0)),
          pl.BlockSpec((1, H), lambda i: (0, 0)),
      ],
      out_specs=[
          pl.BlockSpec((TBLK, H), lambda i: (i, 0)),
          pl.BlockSpec((TBLK, H), lambda i: (i, 0)),
      ],
      out_shape=[jax.ShapeDtypeStruct((N, H), jnp.float32),
                 jax.ShapeDtypeStruct((N, H), jnp.float32)],
  )(acc_lo, acc_hi, h_lo, h_hi, d2, b, Wp, bp)


def kernel(x, edge_index, edge_weight, W1, b1, W2, b2, Wp, bp):
  src2d = edge_index[0].reshape(ROWS, EC)
  dst2d = edge_index[1].reshape(ROWS, EC)
  ew2d = edge_weight.reshape(ROWS, EC)

  d2pad, norm2d = _prep(src2d, dst2d, ew2d)
  d2 = d2pad[:N].reshape(N, 1)

  h1_lo, h1_hi = _mm1(x, W1)
  a1_lo, a1_hi = _msg(h1_lo, h1_hi, src2d, dst2d, norm2d)
  h2_lo, h2_hi = _mid(a1_lo, a1_hi, h1_lo, h1_hi, d2,
                      b1.reshape(1, H), W2)
  a2_lo, a2_hi = _msg(h2_lo, h2_hi, src2d, dst2d, norm2d)
  z, proj = _final(a2_lo, a2_hi, h2_lo, h2_hi, d2,
                   b2.reshape(1, H), Wp, bp.reshape(1, H))
  return (z, proj)


# speedup vs baseline: 2.2712x; 2.2712x over previous
import functools

import numpy as np
import jax
import jax.numpy as jnp
from jax import lax
from jax.experimental import pallas as pl
from jax.experimental.pallas import tpu as pltpu
from jax.experimental.pallas import tpu_sc as plsc

N = 10000
E = 320000
D = 128
H = 128

NP = 10240
EP = 327680
EC = 80
ROWS = EP // EC
RB = 32
NT = 16
NPT = NP // NT
NRT = N // NT
NWB = 624

RPT1 = ROWS // NT
NB1 = RPT1 // RB
RPT2 = ROWS // (2 * NT)
NB2 = RPT2 // RB

_MAGIC = np.int32(0x5F3759DF)


def _rsqrt_newton(x):
  i = plsc.bitcast(x, jnp.int32)
  y = plsc.bitcast(_MAGIC - lax.shift_right_logical(i, 1), jnp.float32)
  for _ in range(3):
    y = y * (1.5 - 0.5 * x * y * y)
  return y


def _prep_body(src2d, dst2d, ew2d, d2_out, norm_out,
               srcbuf, dstbuf, ewbuf, padbuf, idxbuf, degrows, dinvbuf, d2buf,
               dinvv, normbuf, shared_deg, shared_dinv):
  sid = lax.axis_index("s")
  iota16 = lax.iota(jnp.int32, 16)
  zeros16i = jnp.zeros((16,), jnp.int32)
  zeros16f = jnp.zeros((16,), jnp.float32)

  def z_pad(i, _):
    padbuf[i, :] = zeros16f
    return _
  lax.fori_loop(0, EC, z_pad, None)
  for k in range(NPT // EC):
    pltpu.sync_copy(padbuf, shared_deg.at[pl.ds(sid * NPT + k * EC, EC)])
  plsc.subcore_barrier()

  def deg_block(b, _):
    row0 = sid * RPT1 + b * RB
    pltpu.sync_copy(dst2d.at[pl.ds(row0, RB)], dstbuf)
    pltpu.sync_copy(ew2d.at[pl.ds(row0, RB)], ewbuf)

    def deg_chunk(j, _):
      for g in range(EC // 16):
        ewv = ewbuf[j, pl.ds(g * 16, 16)]
        plsc.store_scatter(padbuf, [iota16 + g * 16, zeros16i], ewv)
      pltpu.sync_copy(padbuf, shared_deg.at[dstbuf.at[j]], add=True)
      return _
    lax.fori_loop(0, RB, deg_chunk, None)
    return _
  lax.fori_loop(0, NB1, deg_block, None)
  plsc.subcore_barrier()

  def dinv_batch(t, _):
    base = sid * NPT + t * EC
    for g in range(EC // 16):
      idxbuf[pl.ds(g * 16, 16)] = iota16 + base + g * 16
    pltpu.sync_copy(shared_deg.at[idxbuf], degrows)
    for g in range(EC // 16):
      deg = plsc.load_gather(degrows, [iota16 + g * 16, zeros16i]) + 1.0
      y = _rsqrt_newton(deg)
      dinvbuf[pl.ds(t * EC + g * 16, 16)] = y
      d2buf[pl.ds(t * EC + g * 16, 16)] = y * y
    return _
  lax.fori_loop(0, NPT // EC, dinv_batch, None)
  pltpu.sync_copy(dinvbuf, shared_dinv.at[pl.ds(sid * NPT, NPT)])
  pltpu.sync_copy(d2buf, d2_out.at[pl.ds(sid * NPT, NPT)])
  plsc.subcore_barrier()

  pltpu.sync_copy(shared_dinv, dinvv)

  def norm_block(b, _):
    row0 = sid * RPT1 + b * RB
    pltpu.sync_copy(src2d.at[pl.ds(row0, RB)], srcbuf)
    pltpu.sync_copy(dst2d.at[pl.ds(row0, RB)], dstbuf)
    pltpu.sync_copy(ew2d.at[pl.ds(row0, RB)], ewbuf)

    def norm_chunk(j, _):
      for g in range(EC // 16):
        srcv = srcbuf[j, pl.ds(g * 16, 16)]
        dstv = dstbuf[j, pl.ds(g * 16, 16)]
        ewv = ewbuf[j, pl.ds(g * 16, 16)]
        dsv = plsc.load_gather(dinvv, [srcv])
        ddv = plsc.load_gather(dinvv, [dstv])
        normbuf[j, pl.ds(g * 16, 16)] = dsv * ewv * ddv
      return _
    lax.fori_loop(0, RB, norm_chunk, None)
    pltpu.sync_copy(normbuf, norm_out.at[pl.ds(row0, RB)])
    return _
  lax.fori_loop(0, NB1, norm_block, None)


def _prep(src2d, dst2d, ew2d):
  mesh = plsc.VectorSubcoreMesh(
      core_axis_name="c", subcore_axis_name="s", num_cores=1, num_subcores=NT)
  f = pl.kernel(
      _prep_body,
      out_type=(jax.ShapeDtypeStruct((NP,), jnp.float32),
                jax.ShapeDtypeStruct((ROWS, EC), jnp.float32)),
      mesh=mesh,
      scratch_types=[
          pltpu.VMEM((RB, EC), jnp.int32),
          pltpu.VMEM((RB, EC), jnp.int32),
          pltpu.VMEM((RB, EC), jnp.float32),
          pltpu.VMEM((EC, 16), jnp.float32),
          pltpu.VMEM((EC,), jnp.int32),
          pltpu.VMEM((EC, 16), jnp.float32),
          pltpu.VMEM((NPT,), jnp.float32),
          pltpu.VMEM((NPT,), jnp.float32),
          pltpu.VMEM((NP,), jnp.float32),
          pltpu.VMEM((RB, EC), jnp.float32),
          pltpu.VMEM_SHARED((NP, 16), jnp.float32),
          pltpu.VMEM_SHARED((NP,), jnp.float32),
      ],
      compiler_params=pltpu.CompilerParams(needs_layout_passes=False))
  return f(src2d, dst2d, ew2d)


def _msg_body(h, src2d, dst2d, norm2d, acc_a, acc_b,
              srcbuf, dstbuf, normbuf, rows_v, zbuf, shared_acc):
  cid = lax.axis_index("c")
  sid = lax.axis_index("s")
  iota16 = lax.iota(jnp.int32, 16)
  zeros16f = jnp.zeros((16,), jnp.float32)

  def z_row(i, _):
    for q in range(H // 16):
      zbuf[i, pl.ds(q * 16, 16)] = zeros16f
    return _
  lax.fori_loop(0, 125, z_row, None)
  for k in range(NRT // 125):
    pltpu.sync_copy(zbuf, shared_acc.at[pl.ds(sid * NRT + k * 125, 125)])
  plsc.subcore_barrier()

  def msg_block(b, _):
    row0 = cid * (ROWS // 2) + sid * RPT2 + b * RB
    pltpu.sync_copy(src2d.at[pl.ds(row0, RB)], srcbuf)
    pltpu.sync_copy(dst2d.at[pl.ds(row0, RB)], dstbuf)
    pltpu.sync_copy(norm2d.at[pl.ds(row0, RB)], normbuf)

    def msg_chunk(j, _):
      pltpu.sync_copy(h.at[srcbuf.at[j]], rows_v)

      for g in range(EC // 16):
        rowv = iota16 + g * 16
        normv = normbuf[j, pl.ds(g * 16, 16)]
        for f in range(H):
          colv = jnp.full((16,), f, jnp.int32)
          val = plsc.load_gather(rows_v, [rowv, colv])
          plsc.store_scatter(rows_v, [rowv, colv], val * normv)

      pltpu.sync_copy(rows_v, shared_acc.at[dstbuf.at[j]], add=True)
      return _
    lax.fori_loop(0, RB, msg_chunk, None)
    return _
  lax.fori_loop(0, NB2, msg_block, None)
  plsc.subcore_barrier()

  @pl.when(cid == 0)
  def _w0():
    pltpu.sync_copy(shared_acc.at[pl.ds(sid * NWB, NWB)],
                    acc_a.at[pl.ds(sid * NWB, NWB)])

    @pl.when(sid == NT - 1)
    def _t0():
      pltpu.sync_copy(shared_acc.at[pl.ds(NT * NWB, N - NT * NWB)],
                      acc_a.at[pl.ds(NT * NWB, N - NT * NWB)])

  @pl.when(cid == 1)
  def _w1():
    pltpu.sync_copy(shared_acc.at[pl.ds(sid * NWB, NWB)],
                    acc_b.at[pl.ds(sid * NWB, NWB)])

    @pl.when(sid == NT - 1)
    def _t1():
      pltpu.sync_copy(shared_acc.at[pl.ds(NT * NWB, N - NT * NWB)],
                      acc_b.at[pl.ds(NT * NWB, N - NT * NWB)])


def _msg(h, src2d, dst2d, norm2d):
  mesh = plsc.VectorSubcoreMesh(
      core_axis_name="c", subcore_axis_name="s", num_cores=2, num_subcores=NT)
  f = pl.kernel(
      _msg_body,
      out_type=(jax.ShapeDtypeStruct((N, H), jnp.float32),
                jax.ShapeDtypeStruct((N, H), jnp.float32)),
      mesh=mesh,
      scratch_types=[
          pltpu.VMEM((RB, EC), jnp.int32),
          pltpu.VMEM((RB, EC), jnp.int32),
          pltpu.VMEM((RB, EC), jnp.float32),
          pltpu.VMEM((EC, H), jnp.float32),
          pltpu.VMEM((125, H), jnp.float32),
          pltpu.VMEM_SHARED((N, H), jnp.float32),
      ],
      compiler_params=pltpu.CompilerParams(needs_layout_passes=False))
  return f(h, src2d, dst2d, norm2d)


TBLK = 1000
TNB = N // TBLK


def _mm1_body(x_ref, w_ref, h_ref):
  h_ref[...] = jnp.dot(x_ref[...], w_ref[...],
                       preferred_element_type=jnp.float32)


def _mm1(x, W):
  return pl.pallas_call(
      _mm1_body,
      grid=(TNB,),
      in_specs=[
          pl.BlockSpec((TBLK, D), lambda i: (i, 0)),
          pl.BlockSpec((D, H), lambda i: (0, 0)),
      ],
      out_specs=pl.BlockSpec((TBLK, H), lambda i: (i, 0)),
      out_shape=jax.ShapeDtypeStruct((N, H), jnp.float32),
  )(x, W)


def _mid_body(aa_ref, ab_ref, h_ref, d2_ref, b_ref, w_ref, h2_ref):
  z = (aa_ref[...] + ab_ref[...] + d2_ref[...] * h_ref[...]) + b_ref[...]
  h2_ref[...] = jnp.dot(z, w_ref[...], preferred_element_type=jnp.float32)


def _mid(acc_a, acc_b, h1, d2, b, W):
  return pl.pallas_call(
      _mid_body,
      grid=(TNB,),
      in_specs=[
          pl.BlockSpec((TBLK, H), lambda i: (i, 0)),
          pl.BlockSpec((TBLK, H), lambda i: (i, 0)),
          pl.BlockSpec((TBLK, H), lambda i: (i, 0)),
          pl.BlockSpec((TBLK, 1), lambda i: (i, 0)),
          pl.BlockSpec((1, H), lambda i: (0, 0)),
          pl.BlockSpec((H, H), lambda i: (0, 0)),
      ],
      out_specs=pl.BlockSpec((TBLK, H), lambda i: (i, 0)),
      out_shape=jax.ShapeDtypeStruct((N, H), jnp.float32),
  )(acc_a, acc_b, h1, d2, b, W)


def _final_body(aa_ref, ab_ref, h_ref, d2_ref, b_ref, wp_ref, bp_ref,
                z_ref, p_ref):
  z = (aa_ref[...] + ab_ref[...] + d2_ref[...] * h_ref[...]) + b_ref[...]
  z_ref[...] = z
  p_ref[...] = jnp.dot(z, wp_ref[...],
                       preferred_element_type=jnp.float32) + bp_ref[...]


def _final(acc_a, acc_b, h2, d2, b, Wp, bp):
  return pl.pallas_call(
      _final_body,
      grid=(TNB,),
      in_specs=[
          pl.BlockSpec((TBLK, H), lambda i: (i, 0)),
          pl.BlockSpec((TBLK, H), lambda i: (i, 0)),
          pl.BlockSpec((TBLK, H), lambda i: (i, 0)),
          pl.BlockSpec((TBLK, 1), lambda i: (i, 0)),
          pl.BlockSpec((1, H), lambda i: (0, 0)),
          pl.BlockSpec((H, H), lambda i: (0, 0)),
          pl.BlockSpec((1, H), lambda i: (0, 0)),
      ],
      out_specs=[
          pl.BlockSpec((TBLK, H), lambda i: (i, 0)),
          pl.BlockSpec((TBLK, H), lambda i: (i, 0)),
      ],
      out_shape=[jax.ShapeDtypeStruct((N, H), jnp.float32),
                 jax.ShapeDtypeStruct((N, H), jnp.float32)],
  )(acc_a, acc_b, h2, d2, b, Wp, bp)


def kernel(x, edge_index, edge_weight, W1, b1, W2, b2, Wp, bp):
  padi = jnp.zeros((EP - E,), jnp.int32)
  padf = jnp.zeros((EP - E,), jnp.float32)
  src2d = jnp.concatenate([edge_index[0], padi]).reshape(ROWS, EC)
  dst2d = jnp.concatenate([edge_index[1], padi]).reshape(ROWS, EC)
  ew2d = jnp.concatenate([edge_weight, padf]).reshape(ROWS, EC)

  d2pad, norm2d = _prep(src2d, dst2d, ew2d)
  d2 = d2pad[:N].reshape(N, 1)

  h1 = _mm1(x, W1)
  a1_a, a1_b = _msg(h1, src2d, dst2d, norm2d)
  h2 = _mid(a1_a, a1_b, h1, d2, b1.reshape(1, H), W2)
  a2_a, a2_b = _msg(h2, src2d, dst2d, norm2d)
  z, proj = _final(a2_a, a2_b, h2, d2, b2.reshape(1, H), Wp, bp.reshape(1, H))
  return (z, proj)


# speedup vs baseline: 2.8497x; 1.2547x over previous
import functools

import numpy as np
import jax
import jax.numpy as jnp
from jax import lax
from jax.experimental import pallas as pl
from jax.experimental.pallas import tpu as pltpu
from jax.experimental.pallas import tpu_sc as plsc

N = 10000
E = 320000
D = 128
H = 128

NP = 10240
EP = 327680
EC = 80
ROWS = EP // EC
RB = 32
NT = 16
NPT = NP // NT
NRT = N // NT
NWB = 624

RPT1 = ROWS // NT
NB1 = RPT1 // RB
RPT2 = ROWS // (2 * NT)
NB2 = RPT2 // RB

_MAGIC = np.int32(0x5F3759DF)


def _rsqrt_newton(x):
  i = plsc.bitcast(x, jnp.int32)
  y = plsc.bitcast(_MAGIC - lax.shift_right_logical(i, 1), jnp.float32)
  for _ in range(3):
    y = y * (1.5 - 0.5 * x * y * y)
  return y


def _prep_body(src2d, dst2d, ew2d, d2_out, norm_out,
               srcbuf, dstbuf, ewbuf, padbuf, idxbuf, degrows, dinvbuf, d2buf,
               dinvv, normbuf, shared_deg, shared_dinv):
  sid = lax.axis_index("s")
  iota16 = lax.iota(jnp.int32, 16)
  zeros16i = jnp.zeros((16,), jnp.int32)
  zeros16f = jnp.zeros((16,), jnp.float32)

  def z_pad(i, _):
    padbuf[i, :] = zeros16f
    return _
  lax.fori_loop(0, EC, z_pad, None)
  for k in range(NPT // EC):
    pltpu.sync_copy(padbuf, shared_deg.at[pl.ds(sid * NPT + k * EC, EC)])
  plsc.subcore_barrier()

  def deg_block(b, _):
    row0 = sid * RPT1 + b * RB
    pltpu.sync_copy(dst2d.at[pl.ds(row0, RB)], dstbuf)
    pltpu.sync_copy(ew2d.at[pl.ds(row0, RB)], ewbuf)

    def deg_chunk(j, _):
      for g in range(EC // 16):
        ewv = ewbuf[j, pl.ds(g * 16, 16)]
        plsc.store_scatter(padbuf, [iota16 + g * 16, zeros16i], ewv)
      pltpu.sync_copy(padbuf, shared_deg.at[dstbuf.at[j]], add=True)
      return _
    lax.fori_loop(0, RB, deg_chunk, None)
    return _
  lax.fori_loop(0, NB1, deg_block, None)
  plsc.subcore_barrier()

  def dinv_batch(t, _):
    base = sid * NPT + t * EC
    for g in range(EC // 16):
      idxbuf[pl.ds(g * 16, 16)] = iota16 + base + g * 16
    pltpu.sync_copy(shared_deg.at[idxbuf], degrows)
    for g in range(EC // 16):
      deg = plsc.load_gather(degrows, [iota16 + g * 16, zeros16i]) + 1.0
      y = _rsqrt_newton(deg)
      dinvbuf[pl.ds(t * EC + g * 16, 16)] = y
      d2buf[pl.ds(t * EC + g * 16, 16)] = y * y
    return _
  lax.fori_loop(0, NPT // EC, dinv_batch, None)
  pltpu.sync_copy(dinvbuf, shared_dinv.at[pl.ds(sid * NPT, NPT)])
  pltpu.sync_copy(d2buf, d2_out.at[pl.ds(sid * NPT, NPT)])
  plsc.subcore_barrier()

  pltpu.sync_copy(shared_dinv, dinvv)

  def norm_block(b, _):
    row0 = sid * RPT1 + b * RB
    pltpu.sync_copy(src2d.at[pl.ds(row0, RB)], srcbuf)
    pltpu.sync_copy(dst2d.at[pl.ds(row0, RB)], dstbuf)
    pltpu.sync_copy(ew2d.at[pl.ds(row0, RB)], ewbuf)

    def norm_chunk(j, _):
      for g in range(EC // 16):
        srcv = srcbuf[j, pl.ds(g * 16, 16)]
        dstv = dstbuf[j, pl.ds(g * 16, 16)]
        ewv = ewbuf[j, pl.ds(g * 16, 16)]
        dsv = plsc.load_gather(dinvv, [srcv])
        ddv = plsc.load_gather(dinvv, [dstv])
        normbuf[j, pl.ds(g * 16, 16)] = dsv * ewv * ddv
      return _
    lax.fori_loop(0, RB, norm_chunk, None)
    pltpu.sync_copy(normbuf, norm_out.at[pl.ds(row0, RB)])
    return _
  lax.fori_loop(0, NB1, norm_block, None)


def _prep(src2d, dst2d, ew2d):
  mesh = plsc.VectorSubcoreMesh(
      core_axis_name="c", subcore_axis_name="s", num_cores=1, num_subcores=NT)
  f = pl.kernel(
      _prep_body,
      out_type=(jax.ShapeDtypeStruct((NP,), jnp.float32),
                jax.ShapeDtypeStruct((ROWS, EC), jnp.float32)),
      mesh=mesh,
      scratch_types=[
          pltpu.VMEM((RB, EC), jnp.int32),
          pltpu.VMEM((RB, EC), jnp.int32),
          pltpu.VMEM((RB, EC), jnp.float32),
          pltpu.VMEM((EC, 16), jnp.float32),
          pltpu.VMEM((EC,), jnp.int32),
          pltpu.VMEM((EC, 16), jnp.float32),
          pltpu.VMEM((NPT,), jnp.float32),
          pltpu.VMEM((NPT,), jnp.float32),
          pltpu.VMEM((NP,), jnp.float32),
          pltpu.VMEM((RB, EC), jnp.float32),
          pltpu.VMEM_SHARED((NP, 16), jnp.float32),
          pltpu.VMEM_SHARED((NP,), jnp.float32),
      ],
      compiler_params=pltpu.CompilerParams(needs_layout_passes=False))
  return f(src2d, dst2d, ew2d)


def _msg_body(h, src2d, dst2d, norm2d, acc_a, acc_b,
              srcbuf, dstbuf, normbuf, rows0, rows1, zbuf, shared_acc,
              gsem0, gsem1, ssem0, ssem1):
  cid = lax.axis_index("c")
  sid = lax.axis_index("s")
  iota16 = lax.iota(jnp.int32, 16)
  zeros16f = jnp.zeros((16,), jnp.float32)
  rows = (rows0, rows1)
  gsem = (gsem0, gsem1)
  ssem = (ssem0, ssem1)

  def z_row(i, _):
    for q in range(H // 16):
      zbuf[i, pl.ds(q * 16, 16)] = zeros16f
    return _
  lax.fori_loop(0, 125, z_row, None)
  for k in range(NRT // 125):
    pltpu.sync_copy(zbuf, shared_acc.at[pl.ds(sid * NRT + k * 125, 125)])
  plsc.subcore_barrier()

  def _issue_gather(j, p):
    pltpu.async_copy(h.at[srcbuf.at[j]], rows[p], gsem[p])

  def _wait_gather(p):
    pltpu.make_async_copy(h.at[srcbuf.at[0]], rows[p], gsem[p]).wait()

  def _issue_scatter(j, p):
    pltpu.async_copy(rows[p], shared_acc.at[dstbuf.at[j]], ssem[p], add=True)

  def _wait_scatter(p):
    pltpu.make_async_copy(rows[p], shared_acc.at[dstbuf.at[0]],
                          ssem[p]).wait()

  def _scale(j, p):
    buf = rows[p]
    for g in range(EC // 16):
      rowv = iota16 + g * 16
      normv = normbuf[j, pl.ds(g * 16, 16)]
      for f in range(H):
        colv = jnp.full((16,), f, jnp.int32)
        val = plsc.load_gather(buf, [rowv, colv])
        plsc.store_scatter(buf, [rowv, colv], val * normv)

  def msg_block(b, _):
    row0 = cid * (ROWS // 2) + sid * RPT2 + b * RB
    pltpu.sync_copy(src2d.at[pl.ds(row0, RB)], srcbuf)
    pltpu.sync_copy(dst2d.at[pl.ds(row0, RB)], dstbuf)
    pltpu.sync_copy(norm2d.at[pl.ds(row0, RB)], normbuf)

    _issue_gather(0, 0)

    def msg_chunk(j, _):
      for p in range(2):

        @pl.when(lax.rem(j, 2) == p)
        def _run():
          q = 1 - p
          _wait_gather(p)

          @pl.when(j < RB - 1)
          def _prefetch():
            @pl.when(j >= 1)
            def _drain():
              _wait_scatter(q)
            _issue_gather(j + 1, q)

          _scale(j, p)
          _issue_scatter(j, p)
      return _
    lax.fori_loop(0, RB, msg_chunk, None)
    _wait_scatter(0)
    _wait_scatter(1)
    return _
  lax.fori_loop(0, NB2, msg_block, None)
  plsc.subcore_barrier()

  @pl.when(cid == 0)
  def _w0():
    pltpu.sync_copy(shared_acc.at[pl.ds(sid * NWB, NWB)],
                    acc_a.at[pl.ds(sid * NWB, NWB)])

    @pl.when(sid == NT - 1)
    def _t0():
      pltpu.sync_copy(shared_acc.at[pl.ds(NT * NWB, N - NT * NWB)],
                      acc_a.at[pl.ds(NT * NWB, N - NT * NWB)])

  @pl.when(cid == 1)
  def _w1():
    pltpu.sync_copy(shared_acc.at[pl.ds(sid * NWB, NWB)],
                    acc_b.at[pl.ds(sid * NWB, NWB)])

    @pl.when(sid == NT - 1)
    def _t1():
      pltpu.sync_copy(shared_acc.at[pl.ds(NT * NWB, N - NT * NWB)],
                      acc_b.at[pl.ds(NT * NWB, N - NT * NWB)])


def _msg(h, src2d, dst2d, norm2d):
  mesh = plsc.VectorSubcoreMesh(
      core_axis_name="c", subcore_axis_name="s", num_cores=2, num_subcores=NT)
  f = pl.kernel(
      _msg_body,
      out_type=(jax.ShapeDtypeStruct((N, H), jnp.float32),
                jax.ShapeDtypeStruct((N, H), jnp.float32)),
      mesh=mesh,
      scratch_types=[
          pltpu.VMEM((RB, EC), jnp.int32),
          pltpu.VMEM((RB, EC), jnp.int32),
          pltpu.VMEM((RB, EC), jnp.float32),
          pltpu.VMEM((EC, H), jnp.float32),
          pltpu.VMEM((EC, H), jnp.float32),
          pltpu.VMEM((125, H), jnp.float32),
          pltpu.VMEM_SHARED((N, H), jnp.float32),
          pltpu.SemaphoreType.DMA,
          pltpu.SemaphoreType.DMA,
          pltpu.SemaphoreType.DMA,
          pltpu.SemaphoreType.DMA,
      ],
      compiler_params=pltpu.CompilerParams(needs_layout_passes=False))
  return f(h, src2d, dst2d, norm2d)


TBLK = 1000
TNB = N // TBLK


def _mm1_body(x_ref, w_ref, h_ref):
  h_ref[...] = jnp.dot(x_ref[...], w_ref[...],
                       preferred_element_type=jnp.float32)


def _mm1(x, W):
  return pl.pallas_call(
      _mm1_body,
      grid=(TNB,),
      in_specs=[
          pl.BlockSpec((TBLK, D), lambda i: (i, 0)),
          pl.BlockSpec((D, H), lambda i: (0, 0)),
      ],
      out_specs=pl.BlockSpec((TBLK, H), lambda i: (i, 0)),
      out_shape=jax.ShapeDtypeStruct((N, H), jnp.float32),
  )(x, W)


def _mid_body(aa_ref, ab_ref, h_ref, d2_ref, b_ref, w_ref, h2_ref):
  z = (aa_ref[...] + ab_ref[...] + d2_ref[...] * h_ref[...]) + b_ref[...]
  h2_ref[...] = jnp.dot(z, w_ref[...], preferred_element_type=jnp.float32)


def _mid(acc_a, acc_b, h1, d2, b, W):
  return pl.pallas_call(
      _mid_body,
      grid=(TNB,),
      in_specs=[
          pl.BlockSpec((TBLK, H), lambda i: (i, 0)),
          pl.BlockSpec((TBLK, H), lambda i: (i, 0)),
          pl.BlockSpec((TBLK, H), lambda i: (i, 0)),
          pl.BlockSpec((TBLK, 1), lambda i: (i, 0)),
          pl.BlockSpec((1, H), lambda i: (0, 0)),
          pl.BlockSpec((H, H), lambda i: (0, 0)),
      ],
      out_specs=pl.BlockSpec((TBLK, H), lambda i: (i, 0)),
      out_shape=jax.ShapeDtypeStruct((N, H), jnp.float32),
  )(acc_a, acc_b, h1, d2, b, W)


def _final_body(aa_ref, ab_ref, h_ref, d2_ref, b_ref, wp_ref, bp_ref,
                z_ref, p_ref):
  z = (aa_ref[...] + ab_ref[...] + d2_ref[...] * h_ref[...]) + b_ref[...]
  z_ref[...] = z
  p_ref[...] = jnp.dot(z, wp_ref[...],
                       preferred_element_type=jnp.float32) + bp_ref[...]


def _final(acc_a, acc_b, h2, d2, b, Wp, bp):
  return pl.pallas_call(
      _final_body,
      grid=(TNB,),
      in_specs=[
          pl.BlockSpec((TBLK, H), lambda i: (i, 0)),
          pl.BlockSpec((TBLK, H), lambda i: (i, 0)),
          pl.BlockSpec((TBLK, H), lambda i: (i, 0)),
          pl.BlockSpec((TBLK, 1), lambda i: (i, 0)),
          pl.BlockSpec((1, H), lambda i: (0, 0)),
          pl.BlockSpec((H, H), lambda i: (0, 0)),
          pl.BlockSpec((1, H), lambda i: (0, 0)),
      ],
      out_specs=[
          pl.BlockSpec((TBLK, H), lambda i: (i, 0)),
          pl.BlockSpec((TBLK, H), lambda i: (i, 0)),
      ],
      out_shape=[jax.ShapeDtypeStruct((N, H), jnp.float32),
                 jax.ShapeDtypeStruct((N, H), jnp.float32)],
  )(acc_a, acc_b, h2, d2, b, Wp, bp)


def kernel(x, edge_index, edge_weight, W1, b1, W2, b2, Wp, bp):
  padi = jnp.zeros((EP - E,), jnp.int32)
  padf = jnp.zeros((EP - E,), jnp.float32)
  src2d = jnp.concatenate([edge_index[0], padi]).reshape(ROWS, EC)
  dst2d = jnp.concatenate([edge_index[1], padi]).reshape(ROWS, EC)
  ew2d = jnp.concatenate([edge_weight, padf]).reshape(ROWS, EC)

  d2pad, norm2d = _prep(src2d, dst2d, ew2d)
  d2 = d2pad[:N].reshape(N, 1)

  h1 = _mm1(x, W1)
  a1_a, a1_b = _msg(h1, src2d, dst2d, norm2d)
  h2 = _mid(a1_a, a1_b, h1, d2, b1.reshape(1, H), W2)
  a2_a, a2_b = _msg(h2, src2d, dst2d, norm2d)
  z, proj = _final(a2_a, a2_b, h2, d2, b2.reshape(1, H), Wp, bp.reshape(1, H))
  return (z, proj)


# speedup vs baseline: 2.8713x; 1.0076x over previous
import functools

import numpy as np
import jax
import jax.numpy as jnp
from jax import lax
from jax.experimental import pallas as pl
from jax.experimental.pallas import tpu as pltpu
from jax.experimental.pallas import tpu_sc as plsc

N = 10000
E = 320000
D = 128
H = 128

NP = 10240
EP = 327680
EC = 80
ROWS = EP // EC
RB = 16
NT = 16
NPT = NP // NT
NRT = N // NT
NWB = 624

RPT1 = ROWS // NT
NB1 = RPT1 // RB
RPT2 = ROWS // (2 * NT)
NB2 = RPT2 // RB

_MAGIC = np.int32(0x5F3759DF)


def _rsqrt_newton(x):
  i = plsc.bitcast(x, jnp.int32)
  y = plsc.bitcast(_MAGIC - lax.shift_right_logical(i, 1), jnp.float32)
  for _ in range(3):
    y = y * (1.5 - 0.5 * x * y * y)
  return y


def _prep_body(src2d, dst2d, ew2d, d2_out, norm_out,
               srcbuf, dstbuf, ewbuf, padbuf, idxbuf, degrows, dinvbuf, d2buf,
               dinvv, normbuf, shared_deg, shared_dinv):
  sid = lax.axis_index("s")
  iota16 = lax.iota(jnp.int32, 16)
  zeros16i = jnp.zeros((16,), jnp.int32)
  zeros16f = jnp.zeros((16,), jnp.float32)

  def z_pad(i, _):
    padbuf[i, :] = zeros16f
    return _
  lax.fori_loop(0, EC, z_pad, None)
  for k in range(NPT // EC):
    pltpu.sync_copy(padbuf, shared_deg.at[pl.ds(sid * NPT + k * EC, EC)])
  plsc.subcore_barrier()

  def deg_block(b, _):
    row0 = sid * RPT1 + b * RB
    pltpu.sync_copy(dst2d.at[pl.ds(row0, RB)], dstbuf)
    pltpu.sync_copy(ew2d.at[pl.ds(row0, RB)], ewbuf)

    def deg_chunk(j, _):
      for g in range(EC // 16):
        ewv = ewbuf[j, pl.ds(g * 16, 16)]
        plsc.store_scatter(padbuf, [iota16 + g * 16, zeros16i], ewv)
      pltpu.sync_copy(padbuf, shared_deg.at[dstbuf.at[j]], add=True)
      return _
    lax.fori_loop(0, RB, deg_chunk, None)
    return _
  lax.fori_loop(0, NB1, deg_block, None)
  plsc.subcore_barrier()

  def dinv_batch(t, _):
    base = sid * NPT + t * EC
    for g in range(EC // 16):
      idxbuf[pl.ds(g * 16, 16)] = iota16 + base + g * 16
    pltpu.sync_copy(shared_deg.at[idxbuf], degrows)
    for g in range(EC // 16):
      deg = plsc.load_gather(degrows, [iota16 + g * 16, zeros16i]) + 1.0
      y = _rsqrt_newton(deg)
      dinvbuf[pl.ds(t * EC + g * 16, 16)] = y
      d2buf[pl.ds(t * EC + g * 16, 16)] = y * y
    return _
  lax.fori_loop(0, NPT // EC, dinv_batch, None)
  pltpu.sync_copy(dinvbuf, shared_dinv.at[pl.ds(sid * NPT, NPT)])
  pltpu.sync_copy(d2buf, d2_out.at[pl.ds(sid * NPT, NPT)])
  plsc.subcore_barrier()

  pltpu.sync_copy(shared_dinv, dinvv)

  def norm_block(b, _):
    row0 = sid * RPT1 + b * RB
    pltpu.sync_copy(src2d.at[pl.ds(row0, RB)], srcbuf)
    pltpu.sync_copy(dst2d.at[pl.ds(row0, RB)], dstbuf)
    pltpu.sync_copy(ew2d.at[pl.ds(row0, RB)], ewbuf)

    def norm_chunk(j, _):
      for g in range(EC // 16):
        srcv = srcbuf[j, pl.ds(g * 16, 16)]
        dstv = dstbuf[j, pl.ds(g * 16, 16)]
        ewv = ewbuf[j, pl.ds(g * 16, 16)]
        dsv = plsc.load_gather(dinvv, [srcv])
        ddv = plsc.load_gather(dinvv, [dstv])
        normbuf[j, pl.ds(g * 16, 16)] = dsv * ewv * ddv
      return _
    lax.fori_loop(0, RB, norm_chunk, None)
    pltpu.sync_copy(normbuf, norm_out.at[pl.ds(row0, RB)])
    return _
  lax.fori_loop(0, NB1, norm_block, None)


def _prep(src2d, dst2d, ew2d):
  mesh = plsc.VectorSubcoreMesh(
      core_axis_name="c", subcore_axis_name="s", num_cores=1, num_subcores=NT)
  f = pl.kernel(
      _prep_body,
      out_type=(jax.ShapeDtypeStruct((NP,), jnp.float32),
                jax.ShapeDtypeStruct((ROWS, EC), jnp.float32)),
      mesh=mesh,
      scratch_types=[
          pltpu.VMEM((RB, EC), jnp.int32),
          pltpu.VMEM((RB, EC), jnp.int32),
          pltpu.VMEM((RB, EC), jnp.float32),
          pltpu.VMEM((EC, 16), jnp.float32),
          pltpu.VMEM((EC,), jnp.int32),
          pltpu.VMEM((EC, 16), jnp.float32),
          pltpu.VMEM((NPT,), jnp.float32),
          pltpu.VMEM((NPT,), jnp.float32),
          pltpu.VMEM((NP,), jnp.float32),
          pltpu.VMEM((RB, EC), jnp.float32),
          pltpu.VMEM_SHARED((NP, 16), jnp.float32),
          pltpu.VMEM_SHARED((NP,), jnp.float32),
      ],
      compiler_params=pltpu.CompilerParams(needs_layout_passes=False))
  return f(src2d, dst2d, ew2d)


def _msg_body(h, src2d, dst2d, norm2d, acc_a, acc_b,
              srcbuf, dstbuf, normbuf, rows0, rows1, msg0, msg1,
              shared_acc, gsem0, gsem1, ssem0, ssem1):
  cid = lax.axis_index("c")
  sid = lax.axis_index("s")
  iota16 = lax.iota(jnp.int32, 16)
  zeros16f = jnp.zeros((16,), jnp.float32)
  rows = (rows0, rows1)
  msg = (msg0, msg1)
  gsem = (gsem0, gsem1)
  ssem = (ssem0, ssem1)

  def z_row(i, _):
    for q in range(H // 16):
      msg0[i, pl.ds(q * 16, 16)] = zeros16f
    return _
  lax.fori_loop(0, EC, z_row, None)
  for k in range(NRT // EC):
    pltpu.sync_copy(msg0, shared_acc.at[pl.ds(sid * NRT + k * EC, EC)])
  tail = NRT - (NRT // EC) * EC
  pltpu.sync_copy(
      msg0.at[pl.ds(0, tail)],
      shared_acc.at[pl.ds(sid * NRT + (NRT // EC) * EC, tail)])
  plsc.subcore_barrier()

  def _issue_gather(j, p):
    pltpu.async_copy(h.at[srcbuf.at[j]], rows[p], gsem[p])

  def _wait_gather(p):
    pltpu.make_async_copy(h.at[srcbuf.at[0]], rows[p], gsem[p]).wait()

  def _issue_scatter(j, p):
    pltpu.async_copy(msg[p], shared_acc.at[dstbuf.at[j]], ssem[p], add=True)

  def _wait_scatter(p):
    pltpu.make_async_copy(msg[p], shared_acc.at[dstbuf.at[0]],
                          ssem[p]).wait()

  def _scale(j, p):
    src = rows[p]
    dst = msg[p]
    for g in range(EC // 16):
      rowv = iota16 + g * 16
      normv = normbuf[j, pl.ds(g * 16, 16)]
      for f in range(H):
        colv = jnp.full((16,), f, jnp.int32)
        val = plsc.load_gather(src, [rowv, colv])
        plsc.store_scatter(dst, [rowv, colv], val * normv)

  def msg_block(b, _):
    row0 = cid * (ROWS // 2) + sid * RPT2 + b * RB
    pltpu.sync_copy(src2d.at[pl.ds(row0, RB)], srcbuf)
    pltpu.sync_copy(dst2d.at[pl.ds(row0, RB)], dstbuf)
    pltpu.sync_copy(norm2d.at[pl.ds(row0, RB)], normbuf)

    _issue_gather(0, 0)

    def msg_chunk(j, _):
      for p in range(2):

        @pl.when(lax.rem(j, 2) == p)
        def _run():
          q = 1 - p
          _wait_gather(p)

          @pl.when(j < RB - 1)
          def _prefetch():
            _issue_gather(j + 1, q)

          @pl.when(j >= 2)
          def _drain():
            _wait_scatter(p)

          _scale(j, p)
          _issue_scatter(j, p)
      return _
    lax.fori_loop(0, RB, msg_chunk, None)
    _wait_scatter(0)
    _wait_scatter(1)
    return _
  lax.fori_loop(0, NB2, msg_block, None)
  plsc.subcore_barrier()

  @pl.when(cid == 0)
  def _w0():
    pltpu.sync_copy(shared_acc.at[pl.ds(sid * NWB, NWB)],
                    acc_a.at[pl.ds(sid * NWB, NWB)])

    @pl.when(sid == NT - 1)
    def _t0():
      pltpu.sync_copy(shared_acc.at[pl.ds(NT * NWB, N - NT * NWB)],
                      acc_a.at[pl.ds(NT * NWB, N - NT * NWB)])

  @pl.when(cid == 1)
  def _w1():
    pltpu.sync_copy(shared_acc.at[pl.ds(sid * NWB, NWB)],
                    acc_b.at[pl.ds(sid * NWB, NWB)])

    @pl.when(sid == NT - 1)
    def _t1():
      pltpu.sync_copy(shared_acc.at[pl.ds(NT * NWB, N - NT * NWB)],
                      acc_b.at[pl.ds(NT * NWB, N - NT * NWB)])


def _msg(h, src2d, dst2d, norm2d):
  mesh = plsc.VectorSubcoreMesh(
      core_axis_name="c", subcore_axis_name="s", num_cores=2, num_subcores=NT)
  f = pl.kernel(
      _msg_body,
      out_type=(jax.ShapeDtypeStruct((N, H), jnp.float32),
                jax.ShapeDtypeStruct((N, H), jnp.float32)),
      mesh=mesh,
      scratch_types=[
          pltpu.VMEM((RB, EC), jnp.int32),
          pltpu.VMEM((RB, EC), jnp.int32),
          pltpu.VMEM((RB, EC), jnp.float32),
          pltpu.VMEM((EC, H), jnp.float32),
          pltpu.VMEM((EC, H), jnp.float32),
          pltpu.VMEM((EC, H), jnp.float32),
          pltpu.VMEM((EC, H), jnp.float32),
          pltpu.VMEM_SHARED((N, H), jnp.float32),
          pltpu.SemaphoreType.DMA,
          pltpu.SemaphoreType.DMA,
          pltpu.SemaphoreType.DMA,
          pltpu.SemaphoreType.DMA,
      ],
      compiler_params=pltpu.CompilerParams(needs_layout_passes=False))
  return f(h, src2d, dst2d, norm2d)


TBLK = 1000
TNB = N // TBLK


def _mm1_body(x_ref, w_ref, h_ref):
  h_ref[...] = jnp.dot(x_ref[...], w_ref[...],
                       preferred_element_type=jnp.float32)


def _mm1(x, W):
  return pl.pallas_call(
      _mm1_body,
      grid=(TNB,),
      in_specs=[
          pl.BlockSpec((TBLK, D), lambda i: (i, 0)),
          pl.BlockSpec((D, H), lambda i: (0, 0)),
      ],
      out_specs=pl.BlockSpec((TBLK, H), lambda i: (i, 0)),
      out_shape=jax.ShapeDtypeStruct((N, H), jnp.float32),
  )(x, W)


def _mid_body(aa_ref, ab_ref, h_ref, d2_ref, b_ref, w_ref, h2_ref):
  z = (aa_ref[...] + ab_ref[...] + d2_ref[...] * h_ref[...]) + b_ref[...]
  h2_ref[...] = jnp.dot(z, w_ref[...], preferred_element_type=jnp.float32)


def _mid(acc_a, acc_b, h1, d2, b, W):
  return pl.pallas_call(
      _mid_body,
      grid=(TNB,),
      in_specs=[
          pl.BlockSpec((TBLK, H), lambda i: (i, 0)),
          pl.BlockSpec((TBLK, H), lambda i: (i, 0)),
          pl.BlockSpec((TBLK, H), lambda i: (i, 0)),
          pl.BlockSpec((TBLK, 1), lambda i: (i, 0)),
          pl.BlockSpec((1, H), lambda i: (0, 0)),
          pl.BlockSpec((H, H), lambda i: (0, 0)),
      ],
      out_specs=pl.BlockSpec((TBLK, H), lambda i: (i, 0)),
      out_shape=jax.ShapeDtypeStruct((N, H), jnp.float32),
  )(acc_a, acc_b, h1, d2, b, W)


def _final_body(aa_ref, ab_ref, h_ref, d2_ref, b_ref, wp_ref, bp_ref,
                z_ref, p_ref):
  z = (aa_ref[...] + ab_ref[...] + d2_ref[...] * h_ref[...]) + b_ref[...]
  z_ref[...] = z
  p_ref[...] = jnp.dot(z, wp_ref[...],
                       preferred_element_type=jnp.float32) + bp_ref[...]


def _final(acc_a, acc_b, h2, d2, b, Wp, bp):
  return pl.pallas_call(
      _final_body,
      grid=(TNB,),
      in_specs=[
          pl.BlockSpec((TBLK, H), lambda i: (i, 0)),
          pl.BlockSpec((TBLK, H), lambda i: (i, 0)),
          pl.BlockSpec((TBLK, H), lambda i: (i, 0)),
          pl.BlockSpec((TBLK, 1), lambda i: (i, 0)),
          pl.BlockSpec((1, H), lambda i: (0, 0)),
          pl.BlockSpec((H, H), lambda i: (0, 0)),
          pl.BlockSpec((1, H), lambda i: (0, 0)),
      ],
      out_specs=[
          pl.BlockSpec((TBLK, H), lambda i: (i, 0)),
          pl.BlockSpec((TBLK, H), lambda i: (i, 0)),
      ],
      out_shape=[jax.ShapeDtypeStruct((N, H), jnp.float32),
                 jax.ShapeDtypeStruct((N, H), jnp.float32)],
  )(acc_a, acc_b, h2, d2, b, Wp, bp)


def kernel(x, edge_index, edge_weight, W1, b1, W2, b2, Wp, bp):
  padi = jnp.zeros((EP - E,), jnp.int32)
  padf = jnp.zeros((EP - E,), jnp.float32)
  src2d = jnp.concatenate([edge_index[0], padi]).reshape(ROWS, EC)
  dst2d = jnp.concatenate([edge_index[1], padi]).reshape(ROWS, EC)
  ew2d = jnp.concatenate([edge_weight, padf]).reshape(ROWS, EC)

  d2pad, norm2d = _prep(src2d, dst2d, ew2d)
  d2 = d2pad[:N].reshape(N, 1)

  h1 = _mm1(x, W1)
  a1_a, a1_b = _msg(h1, src2d, dst2d, norm2d)
  h2 = _mid(a1_a, a1_b, h1, d2, b1.reshape(1, H), W2)
  a2_a, a2_b = _msg(h2, src2d, dst2d, norm2d)
  z, proj = _final(a2_a, a2_b, h2, d2, b2.reshape(1, H), Wp, bp.reshape(1, H))
  return (z, proj)


# speedup vs baseline: 8.1864x; 2.8511x over previous
import functools

import numpy as np
import jax
import jax.numpy as jnp
from jax import lax
from jax.experimental import pallas as pl
from jax.experimental.pallas import tpu as pltpu
from jax.experimental.pallas import tpu_sc as plsc

N = 10000
E = 320000
D = 128
H = 128

NP = 10240
EP = 327680
EC = 80
ROWS = EP // EC
RB = 16
NT = 16
NPT = NP // NT
NRT = N // NT
NWB = 624

RPT1 = ROWS // NT
NB1 = RPT1 // RB
RPT2 = ROWS // (2 * NT)
NB2 = RPT2 // RB

_MAGIC = np.int32(0x5F3759DF)
_DIAG_SCATTER = True
_DIAG_SCALE = True


def _rsqrt_newton(x):
  i = plsc.bitcast(x, jnp.int32)
  y = plsc.bitcast(_MAGIC - lax.shift_right_logical(i, 1), jnp.float32)
  for _ in range(3):
    y = y * (1.5 - 0.5 * x * y * y)
  return y


def _prep_body(src2d, dst2d, ew2d, d2_out, norm_out,
               srcbuf, dstbuf, ewbuf, padbuf, idxbuf, degrows, dinvbuf, d2buf,
               dinvv, normbuf, shared_deg, shared_dinv):
  sid = lax.axis_index("s")
  iota16 = lax.iota(jnp.int32, 16)
  zeros16i = jnp.zeros((16,), jnp.int32)
  zeros16f = jnp.zeros((16,), jnp.float32)

  def z_pad(i, _):
    padbuf[i, :] = zeros16f
    return _
  lax.fori_loop(0, EC, z_pad, None)
  for k in range(NPT // EC):
    pltpu.sync_copy(padbuf, shared_deg.at[pl.ds(sid * NPT + k * EC, EC)])
  plsc.subcore_barrier()

  def deg_block(b, _):
    row0 = sid * RPT1 + b * RB
    pltpu.sync_copy(dst2d.at[pl.ds(row0, RB)], dstbuf)
    pltpu.sync_copy(ew2d.at[pl.ds(row0, RB)], ewbuf)

    def deg_chunk(j, _):
      for g in range(EC // 16):
        ewv = ewbuf[j, pl.ds(g * 16, 16)]
        plsc.store_scatter(padbuf, [iota16 + g * 16, zeros16i], ewv)
      pltpu.sync_copy(padbuf, shared_deg.at[dstbuf.at[j]], add=True)
      return _
    lax.fori_loop(0, RB, deg_chunk, None)
    return _
  lax.fori_loop(0, NB1, deg_block, None)
  plsc.subcore_barrier()

  def dinv_batch(t, _):
    base = sid * NPT + t * EC
    for g in range(EC // 16):
      idxbuf[pl.ds(g * 16, 16)] = iota16 + base + g * 16
    pltpu.sync_copy(shared_deg.at[idxbuf], degrows)
    for g in range(EC // 16):
      deg = plsc.load_gather(degrows, [iota16 + g * 16, zeros16i]) + 1.0
      y = _rsqrt_newton(deg)
      dinvbuf[pl.ds(t * EC + g * 16, 16)] = y
      d2buf[pl.ds(t * EC + g * 16, 16)] = y * y
    return _
  lax.fori_loop(0, NPT // EC, dinv_batch, None)
  pltpu.sync_copy(dinvbuf, shared_dinv.at[pl.ds(sid * NPT, NPT)])
  pltpu.sync_copy(d2buf, d2_out.at[pl.ds(sid * NPT, NPT)])
  plsc.subcore_barrier()

  pltpu.sync_copy(shared_dinv, dinvv)

  def norm_block(b, _):
    row0 = sid * RPT1 + b * RB
    pltpu.sync_copy(src2d.at[pl.ds(row0, RB)], srcbuf)
    pltpu.sync_copy(dst2d.at[pl.ds(row0, RB)], dstbuf)
    pltpu.sync_copy(ew2d.at[pl.ds(row0, RB)], ewbuf)

    def norm_chunk(j, _):
      for g in range(EC // 16):
        srcv = srcbuf[j, pl.ds(g * 16, 16)]
        dstv = dstbuf[j, pl.ds(g * 16, 16)]
        ewv = ewbuf[j, pl.ds(g * 16, 16)]
        dsv = plsc.load_gather(dinvv, [srcv])
        ddv = plsc.load_gather(dinvv, [dstv])
        normbuf[j, pl.ds(g * 16, 16)] = dsv * ewv * ddv
      return _
    lax.fori_loop(0, RB, norm_chunk, None)
    pltpu.sync_copy(normbuf, norm_out.at[pl.ds(row0, RB)])
    return _
  lax.fori_loop(0, NB1, norm_block, None)


def _prep(src2d, dst2d, ew2d):
  mesh = plsc.VectorSubcoreMesh(
      core_axis_name="c", subcore_axis_name="s", num_cores=1, num_subcores=NT)
  f = pl.kernel(
      _prep_body,
      out_type=(jax.ShapeDtypeStruct((NP,), jnp.float32),
                jax.ShapeDtypeStruct((ROWS, EC), jnp.float32)),
      mesh=mesh,
      scratch_types=[
          pltpu.VMEM((RB, EC), jnp.int32),
          pltpu.VMEM((RB, EC), jnp.int32),
          pltpu.VMEM((RB, EC), jnp.float32),
          pltpu.VMEM((EC, 16), jnp.float32),
          pltpu.VMEM((EC,), jnp.int32),
          pltpu.VMEM((EC, 16), jnp.float32),
          pltpu.VMEM((NPT,), jnp.float32),
          pltpu.VMEM((NPT,), jnp.float32),
          pltpu.VMEM((NP,), jnp.float32),
          pltpu.VMEM((RB, EC), jnp.float32),
          pltpu.VMEM_SHARED((NP, 16), jnp.float32),
          pltpu.VMEM_SHARED((NP,), jnp.float32),
      ],
      compiler_params=pltpu.CompilerParams(needs_layout_passes=False))
  return f(src2d, dst2d, ew2d)


def _msg_body(h, src2d, dst2d, norm2d, acc_a, acc_b,
              srcbuf, dstbuf, normbuf, rows0, rows1, msg0, msg1,
              shared_acc, gsem0, gsem1, ssem0, ssem1):
  cid = lax.axis_index("c")
  sid = lax.axis_index("s")
  iota16 = lax.iota(jnp.int32, 16)
  zeros16f = jnp.zeros((16,), jnp.float32)
  rows = (rows0, rows1)
  msg = (msg0, msg1)
  gsem = (gsem0, gsem1)
  ssem = (ssem0, ssem1)

  def z_row(i, _):
    for q in range(H // 16):
      msg0[i, pl.ds(q * 16, 16)] = zeros16f
    return _
  lax.fori_loop(0, EC, z_row, None)
  for k in range(NRT // EC):
    pltpu.sync_copy(msg0, shared_acc.at[pl.ds(sid * NRT + k * EC, EC)])
  tail = NRT - (NRT // EC) * EC
  pltpu.sync_copy(
      msg0.at[pl.ds(0, tail)],
      shared_acc.at[pl.ds(sid * NRT + (NRT // EC) * EC, tail)])
  plsc.subcore_barrier()

  def _issue_gather(j, p):
    pltpu.async_copy(h.at[srcbuf.at[j]], rows[p], gsem[p])

  def _wait_gather(p):
    pltpu.make_async_copy(h.at[srcbuf.at[0]], rows[p], gsem[p]).wait()

  def _issue_scatter(j, p):
    pltpu.async_copy(msg[p], shared_acc.at[dstbuf.at[j]], ssem[p], add=True)

  def _wait_scatter(p):
    pltpu.make_async_copy(msg[p], shared_acc.at[dstbuf.at[0]],
                          ssem[p]).wait()

  def _scale(j, p):
    src = rows[p]
    dst = msg[p]
    rowj = jnp.zeros((16,), jnp.int32) + j
    for e in range(EC):
      splat = plsc.load_gather(normbuf, [rowj, jnp.full((16,), e, jnp.int32)])
      for q in range(H // 16):
        dst[e, pl.ds(q * 16, 16)] = src[e, pl.ds(q * 16, 16)] * splat

  def msg_block(b, _):
    row0 = cid * (ROWS // 2) + sid * RPT2 + b * RB
    pltpu.sync_copy(src2d.at[pl.ds(row0, RB)], srcbuf)
    pltpu.sync_copy(dst2d.at[pl.ds(row0, RB)], dstbuf)
    pltpu.sync_copy(norm2d.at[pl.ds(row0, RB)], normbuf)

    _issue_gather(0, 0)

    def msg_chunk(j, _):
      for p in range(2):

        @pl.when(lax.rem(j, 2) == p)
        def _run():
          q = 1 - p
          _wait_gather(p)

          @pl.when(j < RB - 1)
          def _prefetch():
            _issue_gather(j + 1, q)

          if _DIAG_SCATTER:
            @pl.when(j >= 2)
            def _drain():
              _wait_scatter(p)

          if _DIAG_SCALE:
            _scale(j, p)
          if _DIAG_SCATTER:
            _issue_scatter(j, p)
      return _
    lax.fori_loop(0, RB, msg_chunk, None)
    if _DIAG_SCATTER:
      _wait_scatter(0)
      _wait_scatter(1)
    return _
  lax.fori_loop(0, NB2, msg_block, None)
  plsc.subcore_barrier()

  @pl.when(cid == 0)
  def _w0():
    pltpu.sync_copy(shared_acc.at[pl.ds(sid * NWB, NWB)],
                    acc_a.at[pl.ds(sid * NWB, NWB)])

    @pl.when(sid == NT - 1)
    def _t0():
      pltpu.sync_copy(shared_acc.at[pl.ds(NT * NWB, N - NT * NWB)],
                      acc_a.at[pl.ds(NT * NWB, N - NT * NWB)])

  @pl.when(cid == 1)
  def _w1():
    pltpu.sync_copy(shared_acc.at[pl.ds(sid * NWB, NWB)],
                    acc_b.at[pl.ds(sid * NWB, NWB)])

    @pl.when(sid == NT - 1)
    def _t1():
      pltpu.sync_copy(shared_acc.at[pl.ds(NT * NWB, N - NT * NWB)],
                      acc_b.at[pl.ds(NT * NWB, N - NT * NWB)])


def _msg(h, src2d, dst2d, norm2d):
  mesh = plsc.VectorSubcoreMesh(
      core_axis_name="c", subcore_axis_name="s", num_cores=2, num_subcores=NT)
  f = pl.kernel(
      _msg_body,
      out_type=(jax.ShapeDtypeStruct((N, H), jnp.float32),
                jax.ShapeDtypeStruct((N, H), jnp.float32)),
      mesh=mesh,
      scratch_types=[
          pltpu.VMEM((RB, EC), jnp.int32),
          pltpu.VMEM((RB, EC), jnp.int32),
          pltpu.VMEM((RB, EC), jnp.float32),
          pltpu.VMEM((EC, H), jnp.float32),
          pltpu.VMEM((EC, H), jnp.float32),
          pltpu.VMEM((EC, H), jnp.float32),
          pltpu.VMEM((EC, H), jnp.float32),
          pltpu.VMEM_SHARED((N, H), jnp.float32),
          pltpu.SemaphoreType.DMA,
          pltpu.SemaphoreType.DMA,
          pltpu.SemaphoreType.DMA,
          pltpu.SemaphoreType.DMA,
      ],
      compiler_params=pltpu.CompilerParams(needs_layout_passes=False))
  return f(h, src2d, dst2d, norm2d)


TBLK = 1000
TNB = N // TBLK


def _mm1_body(x_ref, w_ref, h_ref):
  h_ref[...] = jnp.dot(x_ref[...], w_ref[...],
                       preferred_element_type=jnp.float32)


def _mm1(x, W):
  return pl.pallas_call(
      _mm1_body,
      grid=(TNB,),
      in_specs=[
          pl.BlockSpec((TBLK, D), lambda i: (i, 0)),
          pl.BlockSpec((D, H), lambda i: (0, 0)),
      ],
      out_specs=pl.BlockSpec((TBLK, H), lambda i: (i, 0)),
      out_shape=jax.ShapeDtypeStruct((N, H), jnp.float32),
  )(x, W)


def _mid_body(aa_ref, ab_ref, h_ref, d2_ref, b_ref, w_ref, h2_ref):
  z = (aa_ref[...] + ab_ref[...] + d2_ref[...] * h_ref[...]) + b_ref[...]
  h2_ref[...] = jnp.dot(z, w_ref[...], preferred_element_type=jnp.float32)


def _mid(acc_a, acc_b, h1, d2, b, W):
  return pl.pallas_call(
      _mid_body,
      grid=(TNB,),
      in_specs=[
          pl.BlockSpec((TBLK, H), lambda i: (i, 0)),
          pl.BlockSpec((TBLK, H), lambda i: (i, 0)),
          pl.BlockSpec((TBLK, H), lambda i: (i, 0)),
          pl.BlockSpec((TBLK, 1), lambda i: (i, 0)),
          pl.BlockSpec((1, H), lambda i: (0, 0)),
          pl.BlockSpec((H, H), lambda i: (0, 0)),
      ],
      out_specs=pl.BlockSpec((TBLK, H), lambda i: (i, 0)),
      out_shape=jax.ShapeDtypeStruct((N, H), jnp.float32),
  )(acc_a, acc_b, h1, d2, b, W)


def _final_body(aa_ref, ab_ref, h_ref, d2_ref, b_ref, wp_ref, bp_ref,
                z_ref, p_ref):
  z = (aa_ref[...] + ab_ref[...] + d2_ref[...] * h_ref[...]) + b_ref[...]
  z_ref[...] = z
  p_ref[...] = jnp.dot(z, wp_ref[...],
                       preferred_element_type=jnp.float32) + bp_ref[...]


def _final(acc_a, acc_b, h2, d2, b, Wp, bp):
  return pl.pallas_call(
      _final_body,
      grid=(TNB,),
      in_specs=[
          pl.BlockSpec((TBLK, H), lambda i: (i, 0)),
          pl.BlockSpec((TBLK, H), lambda i: (i, 0)),
          pl.BlockSpec((TBLK, H), lambda i: (i, 0)),
          pl.BlockSpec((TBLK, 1), lambda i: (i, 0)),
          pl.BlockSpec((1, H), lambda i: (0, 0)),
          pl.BlockSpec((H, H), lambda i: (0, 0)),
          pl.BlockSpec((1, H), lambda i: (0, 0)),
      ],
      out_specs=[
          pl.BlockSpec((TBLK, H), lambda i: (i, 0)),
          pl.BlockSpec((TBLK, H), lambda i: (i, 0)),
      ],
      out_shape=[jax.ShapeDtypeStruct((N, H), jnp.float32),
                 jax.ShapeDtypeStruct((N, H), jnp.float32)],
  )(acc_a, acc_b, h2, d2, b, Wp, bp)


def kernel(x, edge_index, edge_weight, W1, b1, W2, b2, Wp, bp):
  padi = jnp.zeros((EP - E,), jnp.int32)
  padf = jnp.zeros((EP - E,), jnp.float32)
  src2d = jnp.concatenate([edge_index[0], padi]).reshape(ROWS, EC)
  dst2d = jnp.concatenate([edge_index[1], padi]).reshape(ROWS, EC)
  ew2d = jnp.concatenate([edge_weight, padf]).reshape(ROWS, EC)

  d2pad, norm2d = _prep(src2d, dst2d, ew2d)
  d2 = d2pad[:N].reshape(N, 1)

  h1 = _mm1(x, W1)
  a1_a, a1_b = _msg(h1, src2d, dst2d, norm2d)
  h2 = _mid(a1_a, a1_b, h1, d2, b1.reshape(1, H), W2)
  a2_a, a2_b = _msg(h2, src2d, dst2d, norm2d)
  z, proj = _final(a2_a, a2_b, h2, d2, b2.reshape(1, H), Wp, bp.reshape(1, H))
  return (z, proj)


# speedup vs baseline: 8.3850x; 1.0243x over previous
import functools

import numpy as np
import jax
import jax.numpy as jnp
from jax import lax
from jax.experimental import pallas as pl
from jax.experimental.pallas import tpu as pltpu
from jax.experimental.pallas import tpu_sc as plsc

N = 10000
E = 320000
D = 128
H = 128

NP = 10240
EP = 327680
EC = 80
ROWS = EP // EC
RB = 16
NT = 16
NPT = NP // NT
NRT = N // NT
NWB = 624

RPT1 = ROWS // NT
NB1 = RPT1 // RB
RPT2 = ROWS // (2 * NT)
NB2 = RPT2 // RB

_MAGIC = np.int32(0x5F3759DF)


def _rsqrt_newton(x):
  i = plsc.bitcast(x, jnp.int32)
  y = plsc.bitcast(_MAGIC - lax.shift_right_logical(i, 1), jnp.float32)
  for _ in range(3):
    y = y * (1.5 - 0.5 * x * y * y)
  return y


def _prep_body(src2d, dst2d, ew2d, d2_out, norm_out,
               srcbuf, dstbuf, ewbuf, padbuf, idxbuf, degrows, dinvbuf, d2buf,
               dinvv, normbuf, shared_deg, shared_dinv):
  sid = lax.axis_index("s")
  iota16 = lax.iota(jnp.int32, 16)
  zeros16i = jnp.zeros((16,), jnp.int32)
  zeros16f = jnp.zeros((16,), jnp.float32)

  def z_pad(i, _):
    padbuf[i, :] = zeros16f
    return _
  lax.fori_loop(0, EC, z_pad, None)
  for k in range(NPT // EC):
    pltpu.sync_copy(padbuf, shared_deg.at[pl.ds(sid * NPT + k * EC, EC)])
  plsc.subcore_barrier()

  def deg_block(b, _):
    row0 = sid * RPT1 + b * RB
    pltpu.sync_copy(dst2d.at[pl.ds(row0, RB)], dstbuf)
    pltpu.sync_copy(ew2d.at[pl.ds(row0, RB)], ewbuf)

    def deg_chunk(j, _):
      for g in range(EC // 16):
        ewv = ewbuf[j, pl.ds(g * 16, 16)]
        plsc.store_scatter(padbuf, [iota16 + g * 16, zeros16i], ewv)
      pltpu.sync_copy(padbuf, shared_deg.at[dstbuf.at[j]], add=True)
      return _
    lax.fori_loop(0, RB, deg_chunk, None)
    return _
  lax.fori_loop(0, NB1, deg_block, None)
  plsc.subcore_barrier()

  def dinv_batch(t, _):
    base = sid * NPT + t * EC
    for g in range(EC // 16):
      idxbuf[pl.ds(g * 16, 16)] = iota16 + base + g * 16
    pltpu.sync_copy(shared_deg.at[idxbuf], degrows)
    for g in range(EC // 16):
      deg = plsc.load_gather(degrows, [iota16 + g * 16, zeros16i]) + 1.0
      y = _rsqrt_newton(deg)
      dinvbuf[pl.ds(t * EC + g * 16, 16)] = y
      d2buf[pl.ds(t * EC + g * 16, 16)] = y * y
    return _
  lax.fori_loop(0, NPT // EC, dinv_batch, None)
  pltpu.sync_copy(dinvbuf, shared_dinv.at[pl.ds(sid * NPT, NPT)])
  pltpu.sync_copy(d2buf, d2_out.at[pl.ds(sid * NPT, NPT)])
  plsc.subcore_barrier()

  pltpu.sync_copy(shared_dinv, dinvv)

  def norm_block(b, _):
    row0 = sid * RPT1 + b * RB
    pltpu.sync_copy(src2d.at[pl.ds(row0, RB)], srcbuf)
    pltpu.sync_copy(dst2d.at[pl.ds(row0, RB)], dstbuf)
    pltpu.sync_copy(ew2d.at[pl.ds(row0, RB)], ewbuf)

    def norm_chunk(j, _):
      for g in range(EC // 16):
        srcv = srcbuf[j, pl.ds(g * 16, 16)]
        dstv = dstbuf[j, pl.ds(g * 16, 16)]
        ewv = ewbuf[j, pl.ds(g * 16, 16)]
        dsv = plsc.load_gather(dinvv, [srcv])
        ddv = plsc.load_gather(dinvv, [dstv])
        normbuf[j, pl.ds(g * 16, 16)] = dsv * ewv * ddv
      return _
    lax.fori_loop(0, RB, norm_chunk, None)
    pltpu.sync_copy(normbuf, norm_out.at[pl.ds(row0, RB)])
    return _
  lax.fori_loop(0, NB1, norm_block, None)


def _prep(src2d, dst2d, ew2d):
  mesh = plsc.VectorSubcoreMesh(
      core_axis_name="c", subcore_axis_name="s", num_cores=1, num_subcores=NT)
  f = pl.kernel(
      _prep_body,
      out_type=(jax.ShapeDtypeStruct((NP,), jnp.float32),
                jax.ShapeDtypeStruct((ROWS, EC), jnp.float32)),
      mesh=mesh,
      scratch_types=[
          pltpu.VMEM((RB, EC), jnp.int32),
          pltpu.VMEM((RB, EC), jnp.int32),
          pltpu.VMEM((RB, EC), jnp.float32),
          pltpu.VMEM((EC, 16), jnp.float32),
          pltpu.VMEM((EC,), jnp.int32),
          pltpu.VMEM((EC, 16), jnp.float32),
          pltpu.VMEM((NPT,), jnp.float32),
          pltpu.VMEM((NPT,), jnp.float32),
          pltpu.VMEM((NP,), jnp.float32),
          pltpu.VMEM((RB, EC), jnp.float32),
          pltpu.VMEM_SHARED((NP, 16), jnp.float32),
          pltpu.VMEM_SHARED((NP,), jnp.float32),
      ],
      compiler_params=pltpu.CompilerParams(needs_layout_passes=False))
  return f(src2d, dst2d, ew2d)


def _msg_body(h, src2d, dst2d, norm2d, acc_a, acc_b,
              srcbuf, dstbuf, normbuf, rows0, rows1, rows2,
              shared_acc, gsem0, gsem1, gsem2, ssem0, ssem1, ssem2):
  cid = lax.axis_index("c")
  sid = lax.axis_index("s")
  iota16 = lax.iota(jnp.int32, 16)
  zeros16f = jnp.zeros((16,), jnp.float32)
  rows = (rows0, rows1, rows2)
  gsem = (gsem0, gsem1, gsem2)
  ssem = (ssem0, ssem1, ssem2)

  def z_row(i, _):
    for q in range(H // 16):
      rows0[i, pl.ds(q * 16, 16)] = zeros16f
    return _
  lax.fori_loop(0, EC, z_row, None)
  for k in range(NRT // EC):
    pltpu.sync_copy(rows0, shared_acc.at[pl.ds(sid * NRT + k * EC, EC)])
  tail = NRT - (NRT // EC) * EC
  pltpu.sync_copy(
      rows0.at[pl.ds(0, tail)],
      shared_acc.at[pl.ds(sid * NRT + (NRT // EC) * EC, tail)])
  plsc.subcore_barrier()

  def _issue_gather(j, p):
    pltpu.async_copy(h.at[srcbuf.at[j]], rows[p], gsem[p])

  def _wait_gather(p):
    pltpu.make_async_copy(h.at[srcbuf.at[0]], rows[p], gsem[p]).wait()

  def _issue_scatter(j, p):
    pltpu.async_copy(rows[p], shared_acc.at[dstbuf.at[j]], ssem[p], add=True)

  def _wait_scatter(p):
    pltpu.make_async_copy(rows[p], shared_acc.at[dstbuf.at[0]],
                          ssem[p]).wait()

  def _scale(j, p):
    buf = rows[p]
    rowj = jnp.zeros((16,), jnp.int32) + j
    for e in range(EC):
      splat = plsc.load_gather(normbuf, [rowj, jnp.full((16,), e, jnp.int32)])
      for q in range(H // 16):
        buf[e, pl.ds(q * 16, 16)] = buf[e, pl.ds(q * 16, 16)] * splat

  def msg_block(b, _):
    row0 = cid * (ROWS // 2) + sid * RPT2 + b * RB
    pltpu.sync_copy(src2d.at[pl.ds(row0, RB)], srcbuf)
    pltpu.sync_copy(dst2d.at[pl.ds(row0, RB)], dstbuf)
    pltpu.sync_copy(norm2d.at[pl.ds(row0, RB)], normbuf)

    _issue_gather(0, 0)
    _issue_gather(1, 1)

    def msg_chunk(j, _):
      for p in range(3):

        @pl.when(lax.rem(j, 3) == p)
        def _run():
          r = (p + 2) % 3
          _wait_gather(p)

          @pl.when(j <= RB - 3)
          def _prefetch():
            @pl.when(j >= 1)
            def _drain():
              _wait_scatter(r)
            _issue_gather(j + 2, r)

          _scale(j, p)
          _issue_scatter(j, p)
      return _
    lax.fori_loop(0, RB, msg_chunk, None)
    _wait_scatter(0)
    _wait_scatter(1)
    _wait_scatter(2)
    return _
  lax.fori_loop(0, NB2, msg_block, None)
  plsc.subcore_barrier()

  @pl.when(cid == 0)
  def _w0():
    pltpu.sync_copy(shared_acc.at[pl.ds(sid * NWB, NWB)],
                    acc_a.at[pl.ds(sid * NWB, NWB)])

    @pl.when(sid == NT - 1)
    def _t0():
      pltpu.sync_copy(shared_acc.at[pl.ds(NT * NWB, N - NT * NWB)],
                      acc_a.at[pl.ds(NT * NWB, N - NT * NWB)])

  @pl.when(cid == 1)
  def _w1():
    pltpu.sync_copy(shared_acc.at[pl.ds(sid * NWB, NWB)],
                    acc_b.at[pl.ds(sid * NWB, NWB)])

    @pl.when(sid == NT - 1)
    def _t1():
      pltpu.sync_copy(shared_acc.at[pl.ds(NT * NWB, N - NT * NWB)],
                      acc_b.at[pl.ds(NT * NWB, N - NT * NWB)])


def _msg(h, src2d, dst2d, norm2d):
  mesh = plsc.VectorSubcoreMesh(
      core_axis_name="c", subcore_axis_name="s", num_cores=2, num_subcores=NT)
  f = pl.kernel(
      _msg_body,
      out_type=(jax.ShapeDtypeStruct((N, H), jnp.float32),
                jax.ShapeDtypeStruct((N, H), jnp.float32)),
      mesh=mesh,
      scratch_types=[
          pltpu.VMEM((RB, EC), jnp.int32),
          pltpu.VMEM((RB, EC), jnp.int32),
          pltpu.VMEM((RB, EC), jnp.float32),
          pltpu.VMEM((EC, H), jnp.float32),
          pltpu.VMEM((EC, H), jnp.float32),
          pltpu.VMEM((EC, H), jnp.float32),
          pltpu.VMEM_SHARED((N, H), jnp.float32),
          pltpu.SemaphoreType.DMA,
          pltpu.SemaphoreType.DMA,
          pltpu.SemaphoreType.DMA,
          pltpu.SemaphoreType.DMA,
          pltpu.SemaphoreType.DMA,
          pltpu.SemaphoreType.DMA,
      ],
      compiler_params=pltpu.CompilerParams(needs_layout_passes=False))
  return f(h, src2d, dst2d, norm2d)


TBLK = 1000
TNB = N // TBLK


def _mm1_body(x_ref, w_ref, h_ref):
  h_ref[...] = jnp.dot(x_ref[...], w_ref[...],
                       preferred_element_type=jnp.float32)


def _mm1(x, W):
  return pl.pallas_call(
      _mm1_body,
      grid=(TNB,),
      in_specs=[
          pl.BlockSpec((TBLK, D), lambda i: (i, 0)),
          pl.BlockSpec((D, H), lambda i: (0, 0)),
      ],
      out_specs=pl.BlockSpec((TBLK, H), lambda i: (i, 0)),
      out_shape=jax.ShapeDtypeStruct((N, H), jnp.float32),
  )(x, W)


def _mid_body(aa_ref, ab_ref, h_ref, d2_ref, b_ref, w_ref, h2_ref):
  z = (aa_ref[...] + ab_ref[...] + d2_ref[...] * h_ref[...]) + b_ref[...]
  h2_ref[...] = jnp.dot(z, w_ref[...], preferred_element_type=jnp.float32)


def _mid(acc_a, acc_b, h1, d2, b, W):
  return pl.pallas_call(
      _mid_body,
      grid=(TNB,),
      in_specs=[
          pl.BlockSpec((TBLK, H), lambda i: (i, 0)),
          pl.BlockSpec((TBLK, H), lambda i: (i, 0)),
          pl.BlockSpec((TBLK, H), lambda i: (i, 0)),
          pl.BlockSpec((TBLK, 1), lambda i: (i, 0)),
          pl.BlockSpec((1, H), lambda i: (0, 0)),
          pl.BlockSpec((H, H), lambda i: (0, 0)),
      ],
      out_specs=pl.BlockSpec((TBLK, H), lambda i: (i, 0)),
      out_shape=jax.ShapeDtypeStruct((N, H), jnp.float32),
  )(acc_a, acc_b, h1, d2, b, W)


def _final_body(aa_ref, ab_ref, h_ref, d2_ref, b_ref, wp_ref, bp_ref,
                z_ref, p_ref):
  z = (aa_ref[...] + ab_ref[...] + d2_ref[...] * h_ref[...]) + b_ref[...]
  z_ref[...] = z
  p_ref[...] = jnp.dot(z, wp_ref[...],
                       preferred_element_type=jnp.float32) + bp_ref[...]


def _final(acc_a, acc_b, h2, d2, b, Wp, bp):
  return pl.pallas_call(
      _final_body,
      grid=(TNB,),
      in_specs=[
          pl.BlockSpec((TBLK, H), lambda i: (i, 0)),
          pl.BlockSpec((TBLK, H), lambda i: (i, 0)),
          pl.BlockSpec((TBLK, H), lambda i: (i, 0)),
          pl.BlockSpec((TBLK, 1), lambda i: (i, 0)),
          pl.BlockSpec((1, H), lambda i: (0, 0)),
          pl.BlockSpec((H, H), lambda i: (0, 0)),
          pl.BlockSpec((1, H), lambda i: (0, 0)),
      ],
      out_specs=[
          pl.BlockSpec((TBLK, H), lambda i: (i, 0)),
          pl.BlockSpec((TBLK, H), lambda i: (i, 0)),
      ],
      out_shape=[jax.ShapeDtypeStruct((N, H), jnp.float32),
                 jax.ShapeDtypeStruct((N, H), jnp.float32)],
  )(acc_a, acc_b, h2, d2, b, Wp, bp)


def kernel(x, edge_index, edge_weight, W1, b1, W2, b2, Wp, bp):
  padi = jnp.zeros((EP - E,), jnp.int32)
  padf = jnp.zeros((EP - E,), jnp.float32)
  src2d = jnp.concatenate([edge_index[0], padi]).reshape(ROWS, EC)
  dst2d = jnp.concatenate([edge_index[1], padi]).reshape(ROWS, EC)
  ew2d = jnp.concatenate([edge_weight, padf]).reshape(ROWS, EC)

  d2pad, norm2d = _prep(src2d, dst2d, ew2d)
  d2 = d2pad[:N].reshape(N, 1)

  h1 = _mm1(x, W1)
  a1_a, a1_b = _msg(h1, src2d, dst2d, norm2d)
  h2 = _mid(a1_a, a1_b, h1, d2, b1.reshape(1, H), W2)
  a2_a, a2_b = _msg(h2, src2d, dst2d, norm2d)
  z, proj = _final(a2_a, a2_b, h2, d2, b2.reshape(1, H), Wp, bp.reshape(1, H))
  return (z, proj)
